# Initial kernel scaffold; baseline (speedup 1.0000x reference)
import jax
import jax.numpy as jnp
from jax.experimental import pallas as pl


def kernel(x, adj_sparse, eps, W1, b1, W2, b2):
    raise NotImplementedError("write your pallas kernel here")


# speedup vs baseline: 1.0346x; 1.0346x over previous
import jax
import jax.numpy as jnp
from jax.experimental import pallas as pl
from jax.experimental.pallas import tpu as pltpu


def kernel(x, adj_sparse, eps, W1, b1, W2, b2):
    N, D_IN = x.shape
    D_HID = W1.shape[0]
    D_OUT = W2.shape[0]

    BM = 400
    nm = N // BM

    w1t = W1.T
    w2t = W2.T
    b1r = b1.reshape(1, D_HID)
    b2r = b2.reshape(1, D_OUT)
    epsr = eps.reshape(1, 1)

    def body(a_ref, x_ref, eps_ref, w1_ref, b1_ref, w2_ref, b2_ref, o_ref):
        i = pl.program_id(0)
        h = jnp.dot(a_ref[...], x_ref[...],
                    preferred_element_type=jnp.float32)
        xm = x_ref[pl.ds(i * BM, BM), :]
        h = h + (1.0 + eps_ref[0, 0]) * xm
        h1 = jnp.maximum(
            jnp.dot(h, w1_ref[...],
                    preferred_element_type=jnp.float32) + b1_ref[...],
            0.0)
        o_ref[...] = jnp.dot(h1, w2_ref[...],
                             preferred_element_type=jnp.float32) + b2_ref[...]

    return pl.pallas_call(
        body,
        grid=(nm,),
        in_specs=[
            pl.BlockSpec((BM, N), lambda i: (i, 0)),
            pl.BlockSpec((N, D_IN), lambda i: (0, 0)),
            pl.BlockSpec((1, 1), lambda i: (0, 0)),
            pl.BlockSpec((D_IN, D_HID), lambda i: (0, 0)),
            pl.BlockSpec((1, D_HID), lambda i: (0, 0)),
            pl.BlockSpec((D_HID, D_OUT), lambda i: (0, 0)),
            pl.BlockSpec((1, D_OUT), lambda i: (0, 0)),
        ],
        out_specs=pl.BlockSpec((BM, D_OUT), lambda i: (i, 0)),
        out_shape=jax.ShapeDtypeStruct((N, D_OUT), jnp.float32),
        compiler_params=pltpu.CompilerParams(
            dimension_semantics=("arbitrary",)),
    )(adj_sparse, x, epsr, w1t, b1r, w2t, b2r)
